# Initial kernel scaffold; baseline (speedup 1.0000x reference)
import functools
import jax
import jax.numpy as jnp
from jax import lax
from jax.experimental import pallas as pl
from jax.experimental.pallas import tpu as pltpu
from jax.experimental.pallas import tpu_sc as plsc

NC = 2
NS = 16
NW = NC * NS
CH = 128


def _sc_mesh():
    return plsc.VectorSubcoreMesh(core_axis_name="c", subcore_axis_name="s")


def _make_deg_kernel(e_pad, n_pad):
    j_per_w = e_pad // (NW * 16)
    rpt = n_pad // NS

    @functools.partial(
        pl.kernel,
        out_type=jax.ShapeDtypeStruct((NC, n_pad), jnp.float32),
        mesh=_sc_mesh(),
        scratch_types=[
            pltpu.VMEM((e_pad // NW,), jnp.int32),
            pltpu.VMEM((n_pad,), jnp.float32),
            pltpu.VMEM((NS, rpt), jnp.float32),
            pltpu.VMEM_SHARED((NS, n_pad), jnp.float32),
        ],
    )
    def deg_kernel(dst_hbm, out_hbm, dst_v, cnt_v, red_v, stage_sh):
        c = lax.axis_index("c")
        s = lax.axis_index("s")
        w = c * NS + s
        epw = e_pad // NW

        def zero_body(i, _):
            cnt_v[pl.ds(i * 16, 16)] = jnp.zeros((16,), jnp.float32)
            return 0
        lax.fori_loop(0, n_pad // 16, zero_body, 0)

        pltpu.sync_copy(dst_hbm.at[pl.ds(w * epw, epw)], dst_v)

        ones = jnp.ones((16,), jnp.float32)

        def count_body(i, _):
            idx = dst_v[pl.ds(i * 16, 16)]
            plsc.addupdate_scatter(cnt_v, [idx], ones)
            return 0
        lax.fori_loop(0, j_per_w, count_body, 0)

        pltpu.sync_copy(cnt_v, stage_sh.at[s])
        plsc.subcore_barrier()
        pltpu.sync_copy(stage_sh.at[:, pl.ds(s * rpt, rpt)], red_v)

        def red_body(i, _):
            acc = red_v[0, pl.ds(i * 16, 16)]
            for r in range(1, NS):
                acc = acc + red_v[r, pl.ds(i * 16, 16)]
            cnt_v[pl.ds(i * 16, 16)] = acc
            return 0
        lax.fori_loop(0, rpt // 16, red_body, 0)

        pltpu.sync_copy(cnt_v.at[pl.ds(0, rpt)], out_hbm.at[c, pl.ds(s * rpt, rpt)])

    return deg_kernel


def _make_spmm_kernel(e_pad, n_pad, d):
    j_per_w = e_pad // (NW * CH)
    rpt = n_pad // NS
    zr = 64

    @functools.partial(
        pl.kernel,
        out_type=jax.ShapeDtypeStruct((NC, n_pad, d), jnp.float32),
        mesh=_sc_mesh(),
        scratch_types=[
            pltpu.VMEM((j_per_w, CH), jnp.int32),
            pltpu.VMEM((j_per_w, CH), jnp.int32),
            pltpu.VMEM((CH, d), jnp.float32),
            pltpu.VMEM((CH, d), jnp.float32),
            pltpu.VMEM((zr, d), jnp.float32),
            pltpu.VMEM_SHARED((n_pad, d), jnp.float32),
            pltpu.SemaphoreType.DMA,
            pltpu.SemaphoreType.DMA,
        ],
    )
    def spmm_kernel(g_hbm, src_hbm, dst_hbm, out_hbm,
                    src_v, dst_v, rows_a, rows_b, zbuf, acc_sh, sem_a, sem_b):
        c = lax.axis_index("c")
        s = lax.axis_index("s")
        w = c * NS + s

        def zrow(i, _):
            for cc in range(d // 16):
                zbuf[i, pl.ds(cc * 16, 16)] = jnp.zeros((16,), jnp.float32)
            return 0
        lax.fori_loop(0, zr, zrow, 0)
        for k in range(rpt // zr):
            pltpu.sync_copy(zbuf, acc_sh.at[pl.ds(s * rpt + k * zr, zr)])
        plsc.subcore_barrier()

        pltpu.sync_copy(src_hbm.at[pl.ds(w * j_per_w, j_per_w)], src_v)
        pltpu.sync_copy(dst_hbm.at[pl.ds(w * j_per_w, j_per_w)], dst_v)

        pltpu.async_copy(g_hbm.at[src_v.at[0]], rows_a, sem_a)

        def step(j, _):
            even = lax.rem(j, 2) == 0

            @pl.when(even)
            def _():
                pltpu.async_copy(g_hbm.at[src_v.at[j + 1]], rows_b, sem_b)
                pltpu.make_async_copy(g_hbm.at[src_v.at[j]], rows_a, sem_a).wait()
                pltpu.sync_copy(rows_a, acc_sh.at[dst_v.at[j]], add=True)

            @pl.when(jnp.logical_not(even))
            def _():
                pltpu.async_copy(g_hbm.at[src_v.at[j + 1]], rows_a, sem_a)
                pltpu.make_async_copy(g_hbm.at[src_v.at[j]], rows_b, sem_b).wait()
                pltpu.sync_copy(rows_b, acc_sh.at[dst_v.at[j]], add=True)
            return 0

        lax.fori_loop(0, j_per_w - 1, step, 0)

        last = j_per_w - 1
        if last % 2 == 0:
            pltpu.make_async_copy(g_hbm.at[src_v.at[last]], rows_a, sem_a).wait()
            pltpu.sync_copy(rows_a, acc_sh.at[dst_v.at[last]], add=True)
        else:
            pltpu.make_async_copy(g_hbm.at[src_v.at[last]], rows_b, sem_b).wait()
            pltpu.sync_copy(rows_b, acc_sh.at[dst_v.at[last]], add=True)

        plsc.subcore_barrier()
        pltpu.sync_copy(acc_sh.at[pl.ds(s * rpt, rpt)],
                        out_hbm.at[c, pl.ds(s * rpt, rpt)])

    return spmm_kernel


def _tc1_body(cnt_ref, x_ref, w_ref, dinv_ref, g_ref):
    deg = cnt_ref[:, 0:1] + cnt_ref[:, 1:2] + 1.0
    dinv = lax.rsqrt(deg)
    dinv_ref[...] = dinv
    h = jnp.dot(x_ref[...], w_ref[...], preferred_element_type=jnp.float32)
    g_ref[...] = h * dinv


def _tc2_body(acc_ref, g1_ref, dinv_ref, b_ref, w_ref, g2_ref):
    a = acc_ref[0] + acc_ref[1] + g1_ref[...]
    dinv = dinv_ref[...]
    h = jnp.maximum(dinv * a + b_ref[...], 0.0)
    g2_ref[...] = dinv * jnp.dot(h, w_ref[...],
                                 preferred_element_type=jnp.float32)


def _tc3_body(acc_ref, g2_ref, dinv_ref, b_ref, out_ref):
    a = acc_ref[0] + acc_ref[1] + g2_ref[...]
    out_ref[...] = dinv_ref[...] * a + b_ref[...]


def kernel(x, edge_index, edge_attr, W1, b1, W2, b2):
    n, d = x.shape
    e = edge_index.shape[1]

    n_pad = (n + 1 + 255) // 256 * 256
    e_pad = (e + NW * CH - 1) // (NW * CH) * (NW * CH)

    src = edge_index[0]
    dst = edge_index[1]
    pad = e_pad - e
    src_p = jnp.concatenate([src, jnp.full((pad,), n, jnp.int32)])
    dst_p = jnp.concatenate([dst, jnp.full((pad,), n, jnp.int32)])
    src2d = src_p.reshape(e_pad // CH, CH)
    dst2d = dst_p.reshape(e_pad // CH, CH)

    x_pad = jnp.zeros((n_pad, d), x.dtype).at[:n].set(x)
    b1r = b1.reshape(1, d)
    b2r = b2.reshape(1, d)

    deg_k = _make_deg_kernel(e_pad, n_pad)
    spmm_k = _make_spmm_kernel(e_pad, n_pad, d)

    cnt = deg_k(dst_p)
    cnt_t = cnt.T

    r = 1280
    grid = n_pad // r
    row_spec = pl.BlockSpec((r, d), lambda i: (i, 0))
    acc_spec = pl.BlockSpec((NC, r, d), lambda i: (0, i, 0))
    col_spec = pl.BlockSpec((r, 1), lambda i: (i, 0))
    full_spec = pl.BlockSpec((d, d), lambda i: (0, 0))
    bias_spec = pl.BlockSpec((1, d), lambda i: (0, 0))

    dinv, g1 = pl.pallas_call(
        _tc1_body,
        grid=grid,
        in_specs=[pl.BlockSpec((r, NC), lambda i: (i, 0)),
                  row_spec, full_spec],
        out_specs=[col_spec, row_spec],
        out_shape=[jax.ShapeDtypeStruct((n_pad, 1), jnp.float32),
                   jax.ShapeDtypeStruct((n_pad, d), jnp.float32)],
    )(cnt_t, x_pad, W1)

    acc1 = spmm_k(g1, src2d, dst2d)

    g2 = pl.pallas_call(
        _tc2_body,
        grid=grid,
        in_specs=[acc_spec, row_spec, col_spec, bias_spec, full_spec],
        out_specs=row_spec,
        out_shape=jax.ShapeDtypeStruct((n_pad, d), jnp.float32),
    )(acc1, g1, dinv, b1r, W2)

    acc2 = spmm_k(g2, src2d, dst2d)

    out = pl.pallas_call(
        _tc3_body,
        grid=grid,
        in_specs=[acc_spec, row_spec, col_spec, bias_spec],
        out_specs=row_spec,
        out_shape=jax.ShapeDtypeStruct((n_pad, d), jnp.float32),
    )(acc2, g2, dinv, b2r)

    return out[:n]


# speedup vs baseline: 5.6911x; 5.6911x over previous
import functools
import jax
import jax.numpy as jnp
from jax import lax
from jax.experimental import pallas as pl
from jax.experimental.pallas import tpu as pltpu
from jax.experimental.pallas import tpu_sc as plsc

NC = 2
NS = 16
NW = NC * NS
CH = 128


def _sc_mesh():
    return plsc.VectorSubcoreMesh(core_axis_name="c", subcore_axis_name="s")


def _make_deg_kernel(e_pad, n_pad):
    j_per_w = e_pad // (NW * CH)
    rpt = n_pad // NS
    dw = 16

    @functools.partial(
        pl.kernel,
        out_type=jax.ShapeDtypeStruct((NC, n_pad), jnp.float32),
        mesh=_sc_mesh(),
        scratch_types=[
            pltpu.VMEM((j_per_w, CH), jnp.int32),
            pltpu.VMEM((CH,), jnp.float32),
            pltpu.VMEM((rpt,), jnp.float32),
            pltpu.VMEM_SHARED((n_pad,), jnp.float32),
        ],
    )
    def deg_kernel(dst_hbm, out_hbm, dst_v, ones_v, zbuf, deg_sh):
        c = lax.axis_index("c")
        s = lax.axis_index("s")
        w = c * NS + s

        ones = jnp.ones((16,), jnp.float32)
        zeros = jnp.zeros((16,), jnp.float32)

        for i in range(CH // 16):
            ones_v[pl.ds(i * 16, 16)] = ones

        def fill_zero(i, _):
            zbuf[pl.ds(i * 16, 16)] = zeros
            return 0
        lax.fori_loop(0, rpt // 16, fill_zero, 0)

        pltpu.sync_copy(zbuf, deg_sh.at[pl.ds(s * rpt, rpt)])
        pltpu.sync_copy(dst_hbm.at[pl.ds(w * j_per_w, j_per_w)], dst_v)
        plsc.subcore_barrier()

        def body(j, _):
            pltpu.sync_copy(ones_v, deg_sh.at[dst_v.at[j]], add=True)
            return 0
        lax.fori_loop(0, j_per_w, body, 0)

        plsc.subcore_barrier()
        pltpu.sync_copy(deg_sh.at[pl.ds(s * rpt, rpt)],
                        out_hbm.at[c, pl.ds(s * rpt, rpt)])

    return deg_kernel


def _make_spmm_kernel(e_pad, n_pad, d):
    half = n_pad // NC
    j_per_w = e_pad // (NS * CH)
    rpt = half // NS
    zr = 64

    @functools.partial(
        pl.kernel,
        out_type=jax.ShapeDtypeStruct((NC, half, d), jnp.float32),
        mesh=_sc_mesh(),
        scratch_types=[
            pltpu.VMEM((j_per_w, CH), jnp.int32),
            pltpu.VMEM((j_per_w, CH), jnp.int32),
            pltpu.VMEM((CH, d), jnp.float32),
            pltpu.VMEM((CH, d), jnp.float32),
            pltpu.VMEM((zr, d), jnp.float32),
            pltpu.VMEM_SHARED((half + 8, d), jnp.float32),
            pltpu.SemaphoreType.DMA,
            pltpu.SemaphoreType.DMA,
        ],
    )
    def spmm_kernel(g_hbm, src_hbm, dst_hbm, out_hbm,
                    src_v, dst_v, rows_a, rows_b, zbuf, acc_sh, sem_a, sem_b):
        c = lax.axis_index("c")
        s = lax.axis_index("s")

        def zrow(i, _):
            for cc in range(d // 16):
                zbuf[i, pl.ds(cc * 16, 16)] = jnp.zeros((16,), jnp.float32)
            return 0
        lax.fori_loop(0, zr, zrow, 0)
        for k in range(rpt // zr):
            pltpu.sync_copy(zbuf, acc_sh.at[pl.ds(s * rpt + k * zr, zr)])
        plsc.subcore_barrier()

        pltpu.sync_copy(src_hbm.at[pl.ds(s * j_per_w, j_per_w)], src_v)
        pltpu.sync_copy(dst_hbm.at[c, pl.ds(s * j_per_w, j_per_w)], dst_v)

        def step(j, _):
            pltpu.async_copy(g_hbm.at[src_v.at[j]], rows_a, sem_a).wait()
            pltpu.sync_copy(rows_a, acc_sh.at[dst_v.at[j]], add=True)
            return 0

        lax.fori_loop(0, j_per_w, step, 0)
        del rows_b, sem_b

        plsc.subcore_barrier()
        pltpu.sync_copy(acc_sh.at[pl.ds(s * rpt, rpt)],
                        out_hbm.at[c, pl.ds(s * rpt, rpt)])

    return spmm_kernel


def _tc1_body(cnt_ref, x_ref, w_ref, dinv_ref, g_ref):
    deg = cnt_ref[:, 0:1] + cnt_ref[:, 1:2] + 1.0
    dinv = lax.rsqrt(deg)
    dinv_ref[...] = dinv
    h = jnp.dot(x_ref[...], w_ref[...], preferred_element_type=jnp.float32)
    g_ref[...] = h * dinv


def _tc2_body(acc_ref, g1_ref, dinv_ref, b_ref, w_ref, g2_ref):
    a = acc_ref[...] + g1_ref[...]
    dinv = dinv_ref[...]
    h = jnp.maximum(dinv * a + b_ref[...], 0.0)
    g2_ref[...] = dinv * jnp.dot(h, w_ref[...],
                                 preferred_element_type=jnp.float32)


def _tc3_body(acc_ref, g2_ref, dinv_ref, b_ref, out_ref):
    a = acc_ref[...] + g2_ref[...]
    out_ref[...] = dinv_ref[...] * a + b_ref[...]


def kernel(x, edge_index, edge_attr, W1, b1, W2, b2):
    n, d = x.shape
    e = edge_index.shape[1]

    n_pad = (n + 1 + 255) // 256 * 256
    half = n_pad // NC
    e_pad = (e + NS * CH * 8 - 1) // (NS * CH * 8) * (NS * CH * 8)

    src = edge_index[0]
    dst = edge_index[1]
    pad = e_pad - e
    src_p = jnp.concatenate([src, jnp.full((pad,), n, jnp.int32)])
    dst_p = jnp.concatenate([dst, jnp.full((pad,), n, jnp.int32)])
    src2d = src_p.reshape(e_pad // CH, CH)
    dst2d = dst_p.reshape(e_pad // CH, CH)
    dst_cores = []
    for c in range(NC):
        lo = c * half
        owned = (dst_p >= lo) & (dst_p < lo + half)
        dst_cores.append(jnp.where(owned, dst_p - lo, half))
    dst3d = jnp.stack(dst_cores).reshape(NC, e_pad // CH, CH)

    x_pad = jnp.zeros((n_pad, d), x.dtype).at[:n].set(x)
    b1r = b1.reshape(1, d)
    b2r = b2.reshape(1, d)

    deg_k = _make_deg_kernel(e_pad, n_pad)
    spmm_k = _make_spmm_kernel(e_pad, n_pad, d)

    BISECT_JNP_DEG = False
    if BISECT_JNP_DEG:
        cnt0 = jnp.zeros((n_pad,), jnp.float32).at[dst].add(1.0)
        cnt_t = jnp.stack([cnt0, jnp.zeros_like(cnt0)], axis=1)
    else:
        cnt = deg_k(dst2d)
        cnt_t = cnt.T

    r = 1280
    grid = n_pad // r
    row_spec = pl.BlockSpec((r, d), lambda i: (i, 0))
    col_spec = pl.BlockSpec((r, 1), lambda i: (i, 0))
    full_spec = pl.BlockSpec((d, d), lambda i: (0, 0))
    bias_spec = pl.BlockSpec((1, d), lambda i: (0, 0))

    dinv, g1 = pl.pallas_call(
        _tc1_body,
        grid=grid,
        in_specs=[pl.BlockSpec((r, NC), lambda i: (i, 0)),
                  row_spec, full_spec],
        out_specs=[col_spec, row_spec],
        out_shape=[jax.ShapeDtypeStruct((n_pad, 1), jnp.float32),
                   jax.ShapeDtypeStruct((n_pad, d), jnp.float32)],
    )(cnt_t, x_pad, W1)

    acc1 = spmm_k(g1, src2d, dst3d).reshape(n_pad, d)

    g2 = pl.pallas_call(
        _tc2_body,
        grid=grid,
        in_specs=[row_spec, row_spec, col_spec, bias_spec, full_spec],
        out_specs=row_spec,
        out_shape=jax.ShapeDtypeStruct((n_pad, d), jnp.float32),
    )(acc1, g1, dinv, b1r, W2)

    acc2 = spmm_k(g2, src2d, dst3d).reshape(n_pad, d)

    out = pl.pallas_call(
        _tc3_body,
        grid=grid,
        in_specs=[row_spec, row_spec, col_spec, bias_spec],
        out_specs=row_spec,
        out_shape=jax.ShapeDtypeStruct((n_pad, d), jnp.float32),
    )(acc2, g2, dinv, b2r)

    return out[:n]


# speedup vs baseline: 6.0311x; 1.0598x over previous
import functools
import jax
import jax.numpy as jnp
from jax import lax
from jax.experimental import pallas as pl
from jax.experimental.pallas import tpu as pltpu
from jax.experimental.pallas import tpu_sc as plsc

NC = 2
NS = 16
NW = NC * NS
CH = 128


def _sc_mesh():
    return plsc.VectorSubcoreMesh(core_axis_name="c", subcore_axis_name="s")


def _make_deg_kernel(e_pad, n_pad):
    j_per_w = e_pad // (NW * CH)
    rpt = n_pad // NS
    dw = 16

    @functools.partial(
        pl.kernel,
        out_type=jax.ShapeDtypeStruct((NC, n_pad), jnp.float32),
        mesh=_sc_mesh(),
        scratch_types=[
            pltpu.VMEM((j_per_w, CH), jnp.int32),
            pltpu.VMEM((CH,), jnp.float32),
            pltpu.VMEM((rpt,), jnp.float32),
            pltpu.VMEM_SHARED((n_pad,), jnp.float32),
        ],
    )
    def deg_kernel(dst_hbm, out_hbm, dst_v, ones_v, zbuf, deg_sh):
        c = lax.axis_index("c")
        s = lax.axis_index("s")
        w = c * NS + s

        ones = jnp.ones((16,), jnp.float32)
        zeros = jnp.zeros((16,), jnp.float32)

        for i in range(CH // 16):
            ones_v[pl.ds(i * 16, 16)] = ones

        def fill_zero(i, _):
            zbuf[pl.ds(i * 16, 16)] = zeros
            return 0
        lax.fori_loop(0, rpt // 16, fill_zero, 0)

        pltpu.sync_copy(zbuf, deg_sh.at[pl.ds(s * rpt, rpt)])
        pltpu.sync_copy(dst_hbm.at[pl.ds(w * j_per_w, j_per_w)], dst_v)
        plsc.subcore_barrier()

        def body(j, _):
            pltpu.sync_copy(ones_v, deg_sh.at[dst_v.at[j]], add=True)
            return 0
        lax.fori_loop(0, j_per_w, body, 0)

        plsc.subcore_barrier()
        pltpu.sync_copy(deg_sh.at[pl.ds(s * rpt, rpt)],
                        out_hbm.at[c, pl.ds(s * rpt, rpt)])

    return deg_kernel


def _make_spmm_kernel(e_pad, n_pad, d):
    half = n_pad // NC
    j_per_w = e_pad // (NS * CH)
    rpt = half // NS
    zr = 64

    @functools.partial(
        pl.kernel,
        out_type=jax.ShapeDtypeStruct((NC, half, d), jnp.float32),
        mesh=_sc_mesh(),
        scratch_types=[
            pltpu.VMEM((j_per_w, CH), jnp.int32),
            pltpu.VMEM((j_per_w, CH), jnp.int32),
            pltpu.VMEM((CH, d), jnp.float32),
            pltpu.VMEM((CH, d), jnp.float32),
            pltpu.VMEM((zr, d), jnp.float32),
            pltpu.VMEM_SHARED((half + 8, d), jnp.float32),
            pltpu.SemaphoreType.DMA,
            pltpu.SemaphoreType.DMA,
        ],
    )
    def spmm_kernel(g_hbm, src_hbm, dst_hbm, out_hbm,
                    src_v, dst_v, rows_a, rows_b, zbuf, acc_sh, sem_a, sem_b):
        c = lax.axis_index("c")
        s = lax.axis_index("s")

        def zrow(i, _):
            for cc in range(d // 16):
                zbuf[i, pl.ds(cc * 16, 16)] = jnp.zeros((16,), jnp.float32)
            return 0
        lax.fori_loop(0, zr, zrow, 0)
        for k in range(rpt // zr):
            pltpu.sync_copy(zbuf, acc_sh.at[pl.ds(s * rpt + k * zr, zr)])
        plsc.subcore_barrier()

        pltpu.sync_copy(src_hbm.at[pl.ds(s * j_per_w, j_per_w)], src_v)
        pltpu.sync_copy(dst_hbm.at[c, pl.ds(s * j_per_w, j_per_w)], dst_v)

        pltpu.async_copy(g_hbm.at[src_v.at[0]], rows_a, sem_a)

        def step(j, _):
            even = lax.rem(j, 2) == 0

            @pl.when(even)
            def _():
                pltpu.async_copy(g_hbm.at[src_v.at[j + 1]], rows_b, sem_b)
                pltpu.make_async_copy(g_hbm.at[src_v.at[j]], rows_a, sem_a).wait()
                pltpu.sync_copy(rows_a, acc_sh.at[dst_v.at[j]], add=True)

            @pl.when(jnp.logical_not(even))
            def _():
                pltpu.async_copy(g_hbm.at[src_v.at[j + 1]], rows_a, sem_a)
                pltpu.make_async_copy(g_hbm.at[src_v.at[j]], rows_b, sem_b).wait()
                pltpu.sync_copy(rows_b, acc_sh.at[dst_v.at[j]], add=True)
            return 0

        lax.fori_loop(0, j_per_w - 1, step, 0)

        last = j_per_w - 1
        if last % 2 == 0:
            pltpu.make_async_copy(g_hbm.at[src_v.at[last]], rows_a, sem_a).wait()
            pltpu.sync_copy(rows_a, acc_sh.at[dst_v.at[last]], add=True)
        else:
            pltpu.make_async_copy(g_hbm.at[src_v.at[last]], rows_b, sem_b).wait()
            pltpu.sync_copy(rows_b, acc_sh.at[dst_v.at[last]], add=True)

        plsc.subcore_barrier()
        pltpu.sync_copy(acc_sh.at[pl.ds(s * rpt, rpt)],
                        out_hbm.at[c, pl.ds(s * rpt, rpt)])

    return spmm_kernel


def _tc1_body(cnt_ref, x_ref, w_ref, dinv_ref, g_ref):
    deg = cnt_ref[:, 0:1] + cnt_ref[:, 1:2] + 1.0
    dinv = lax.rsqrt(deg)
    dinv_ref[...] = dinv
    h = jnp.dot(x_ref[...], w_ref[...], preferred_element_type=jnp.float32)
    g_ref[...] = h * dinv


def _tc2_body(acc_ref, g1_ref, dinv_ref, b_ref, w_ref, g2_ref):
    a = acc_ref[...] + g1_ref[...]
    dinv = dinv_ref[...]
    h = jnp.maximum(dinv * a + b_ref[...], 0.0)
    g2_ref[...] = dinv * jnp.dot(h, w_ref[...],
                                 preferred_element_type=jnp.float32)


def _tc3_body(acc_ref, g2_ref, dinv_ref, b_ref, out_ref):
    a = acc_ref[...] + g2_ref[...]
    out_ref[...] = dinv_ref[...] * a + b_ref[...]


def kernel(x, edge_index, edge_attr, W1, b1, W2, b2):
    n, d = x.shape
    e = edge_index.shape[1]

    n_pad = (n + 1 + 255) // 256 * 256
    half = n_pad // NC
    e_pad = (e + NS * CH * 8 - 1) // (NS * CH * 8) * (NS * CH * 8)

    src = edge_index[0]
    dst = edge_index[1]
    pad = e_pad - e
    src_p = jnp.concatenate([src, jnp.full((pad,), n, jnp.int32)])
    dst_p = jnp.concatenate([dst, jnp.full((pad,), n, jnp.int32)])
    src2d = src_p.reshape(e_pad // CH, CH)
    dst2d = dst_p.reshape(e_pad // CH, CH)
    dst_cores = []
    for c in range(NC):
        lo = c * half
        owned = (dst_p >= lo) & (dst_p < lo + half)
        dst_cores.append(jnp.where(owned, dst_p - lo, half))
    dst3d = jnp.stack(dst_cores).reshape(NC, e_pad // CH, CH)

    x_pad = jnp.zeros((n_pad, d), x.dtype).at[:n].set(x)
    b1r = b1.reshape(1, d)
    b2r = b2.reshape(1, d)

    deg_k = _make_deg_kernel(e_pad, n_pad)
    spmm_k = _make_spmm_kernel(e_pad, n_pad, d)

    BISECT_JNP_DEG = False
    if BISECT_JNP_DEG:
        cnt0 = jnp.zeros((n_pad,), jnp.float32).at[dst].add(1.0)
        cnt_t = jnp.stack([cnt0, jnp.zeros_like(cnt0)], axis=1)
    else:
        cnt = deg_k(dst2d)
        cnt_t = cnt.T

    r = 1280
    grid = n_pad // r
    row_spec = pl.BlockSpec((r, d), lambda i: (i, 0))
    col_spec = pl.BlockSpec((r, 1), lambda i: (i, 0))
    full_spec = pl.BlockSpec((d, d), lambda i: (0, 0))
    bias_spec = pl.BlockSpec((1, d), lambda i: (0, 0))

    dinv, g1 = pl.pallas_call(
        _tc1_body,
        grid=grid,
        in_specs=[pl.BlockSpec((r, NC), lambda i: (i, 0)),
                  row_spec, full_spec],
        out_specs=[col_spec, row_spec],
        out_shape=[jax.ShapeDtypeStruct((n_pad, 1), jnp.float32),
                   jax.ShapeDtypeStruct((n_pad, d), jnp.float32)],
    )(cnt_t, x_pad, W1)

    acc1 = spmm_k(g1, src2d, dst3d).reshape(n_pad, d)

    g2 = pl.pallas_call(
        _tc2_body,
        grid=grid,
        in_specs=[row_spec, row_spec, col_spec, bias_spec, full_spec],
        out_specs=row_spec,
        out_shape=jax.ShapeDtypeStruct((n_pad, d), jnp.float32),
    )(acc1, g1, dinv, b1r, W2)

    acc2 = spmm_k(g2, src2d, dst3d).reshape(n_pad, d)

    out = pl.pallas_call(
        _tc3_body,
        grid=grid,
        in_specs=[row_spec, row_spec, col_spec, bias_spec],
        out_specs=row_spec,
        out_shape=jax.ShapeDtypeStruct((n_pad, d), jnp.float32),
    )(acc2, g2, dinv, b2r)

    return out[:n]
